# baseline (speedup 1.0000x reference)
import functools

import jax
import jax.numpy as jnp
from jax import lax
from jax.experimental import pallas as pl
from jax.experimental.pallas import tpu as pltpu
from jax.experimental.pallas import tpu_sc as plsc

N = 10000
E = 320000
D = 128
H1 = 32
H1A = 48
H2 = 64
H2A = 80
NPAD = 10240
NW = 32
ECH = 80
NCH = 125
RING = 5
RPS = NPAD // 16


_BCAST_DNUMS = lax.GatherDimensionNumbers(
    offset_dims=(), collapsed_slice_dims=(0,), start_index_map=(0,))


def _elu(v):
    return jnp.where(v > 0, v, jnp.exp(jnp.minimum(v, 0.0)) - 1.0)


def _tc_a_body(x_ref, w1a_ref, b1a_ref, w1b_ref, w1c_ref, b1c_ref,
               a_ref, b_ref, c_ref):
    x = x_ref[...]
    a = jnp.dot(x, w1a_ref[...], preferred_element_type=jnp.float32) + b1a_ref[...]
    a_ref[...] = jnp.concatenate(
        [a, jnp.ones((N, 1), jnp.float32),
         jnp.zeros((N, H1A - H1 - 1), jnp.float32)], axis=1)
    b_ref[...] = jnp.dot(x, w1b_ref[...], preferred_element_type=jnp.float32)
    c_ref[...] = jnp.dot(x, w1c_ref[...], preferred_element_type=jnp.float32) + b1c_ref[...]


def _tc_b_body(p_ref, b1_ref, c1_ref, w2a_ref, b2a_ref, w2b_ref, w2c_ref,
               b2c_ref, a2_ref, b2_ref, c2_ref, wdeg_ref):
    agg = p_ref[0, pl.ds(0, N), pl.ds(0, H1)] + p_ref[1, pl.ds(0, N), pl.ds(0, H1)]
    wdeg = p_ref[0, pl.ds(0, N), pl.ds(H1, 1)] + p_ref[1, pl.ds(0, N), pl.ds(H1, 1)]
    h1 = _elu(agg - b1_ref[...] * wdeg + c1_ref[...])
    a2 = jnp.dot(h1, w2a_ref[...], preferred_element_type=jnp.float32) + b2a_ref[...]
    a2_ref[...] = jnp.concatenate(
        [a2, jnp.zeros((N, H2A - H2), jnp.float32)], axis=1)
    b2_ref[...] = jnp.dot(h1, w2b_ref[...], preferred_element_type=jnp.float32)
    c2_ref[...] = jnp.dot(h1, w2c_ref[...], preferred_element_type=jnp.float32) + b2c_ref[...]
    wdeg_ref[...] = wdeg


def _tc_c_body(p_ref, b2_ref, c2_ref, wdeg_ref, fc1w_ref, fc1b_ref,
               fc2w_ref, fc2b_ref, out_ref):
    agg = (p_ref[0, pl.ds(0, N), pl.ds(0, H2)]
           + p_ref[1, pl.ds(0, N), pl.ds(0, H2)])
    h2 = _elu(agg - b2_ref[...] * wdeg_ref[...] + c2_ref[...])
    h3 = _elu(jnp.dot(h2, fc1w_ref[...], preferred_element_type=jnp.float32) + fc1b_ref[...])
    lg = jnp.dot(h3, fc2w_ref[...], preferred_element_type=jnp.float32) + fc2b_ref[...]
    m = jnp.max(lg, axis=1, keepdims=True)
    out_ref[...] = lg - m - jnp.log(jnp.sum(jnp.exp(lg - m), axis=1, keepdims=True))


def _run_tc(body, out_shapes, *args):
    return pl.pallas_call(body, out_shape=out_shapes)(*args)


@functools.lru_cache(maxsize=None)
def _make_sc_edge(h, hs):
    mesh = plsc.VectorSubcoreMesh(core_axis_name="c", subcore_axis_name="s")

    @functools.partial(
        pl.kernel,
        out_type=jax.ShapeDtypeStruct((2, NPAD, h), jnp.float32),
        mesh=mesh,
        compiler_params=pltpu.CompilerParams(use_tc_tiling_on_sc=False),
        scratch_types=[
            pltpu.VMEM((NCH, ECH), jnp.int32),
            pltpu.VMEM((NCH, ECH), jnp.int32),
            pltpu.VMEM((NCH, ECH), jnp.float32),
            [pltpu.VMEM((ECH, h), jnp.float32)] * RING,
            pltpu.VMEM_SHARED((NPAD, h), jnp.float32),
            [pltpu.SemaphoreType.DMA] * RING,
            [pltpu.SemaphoreType.DMA] * RING,
        ],
    )
    def sc_edge(a_hbm, ei_hbm, ews_hbm, out_hbm,
                src_v, dst_v, ew_v, rows, agg_sh, gsems, ssems):
        c = lax.axis_index("c")
        s = lax.axis_index("s")
        wid = c * 16 + s

        pltpu.sync_copy(ei_hbm.at[0, wid], src_v)
        pltpu.sync_copy(ei_hbm.at[1, wid], dst_v)
        pltpu.sync_copy(ews_hbm.at[wid], ew_v)

        zf = jnp.zeros((16,), jnp.float32)

        def zero_body(r, carry):
            for j in range(h // 16):
                rows[0][r, pl.ds(j * 16, 16)] = zf
            return carry

        lax.fori_loop(0, ECH, zero_body, 0)
        for k in range(RPS // ECH):
            pltpu.sync_copy(rows[0], agg_sh.at[pl.ds(s * RPS + k * ECH, ECH)])
        plsc.subcore_barrier()

        def scale(buf, ch):
            def group_body(g, c2):
                vew = ew_v[ch, pl.ds(g * 16, 16)]
                for r16 in range(16):
                    idx = jnp.full((16, 1), r16, jnp.int32)
                    wv = lax.gather(vew, idx, _BCAST_DNUMS, (1,),
                                    mode=lax.GatherScatterMode.PROMISE_IN_BOUNDS)
                    r = g * 16 + r16
                    for j in range(hs // 16):
                        sl = pl.ds(j * 16, 16)
                        buf[r, sl] = buf[r, sl] * wv
                return c2

            lax.fori_loop(0, ECH // 16, group_body, 0)

        pltpu.async_copy(a_hbm.at[src_v.at[0]], rows[0], gsems[0])
        pltpu.async_copy(a_hbm.at[src_v.at[1]], rows[1], gsems[1])

        def ring_body(p, carry):
            for b in range(RING):
                ch = RING * p + b
                bn = (b + 2) % RING

                @pl.when(ch + 2 - RING >= 0)
                def _():
                    pltpu.make_async_copy(
                        rows[bn], agg_sh.at[dst_v.at[ch + 2 - RING]], ssems[bn]
                    ).wait()

                @pl.when(ch + 2 < NCH)
                def _():
                    pltpu.async_copy(
                        a_hbm.at[src_v.at[ch + 2]], rows[bn], gsems[bn])

                pltpu.make_async_copy(
                    a_hbm.at[src_v.at[ch]], rows[b], gsems[b]).wait()
                scale(rows[b], ch)
                pltpu.async_copy(
                    rows[b], agg_sh.at[dst_v.at[ch]], ssems[b], add=True)
            return carry

        lax.fori_loop(0, NCH // RING, ring_body, 0)

        for ch in range(NCH - RING + 2, NCH):
            pltpu.make_async_copy(
                rows[ch % RING], agg_sh.at[dst_v.at[ch]], ssems[ch % RING]
            ).wait()
        plsc.subcore_barrier()

        pltpu.sync_copy(agg_sh.at[pl.ds(s * RPS, RPS)],
                        out_hbm.at[c, pl.ds(s * RPS, RPS)])

    return sc_edge


def kernel(x, edge_index, edge_attr, w1a, b1a, w1b, w1c, b1c,
           w2a, b2a, w2b, w2c, b2c, fc1_w, fc1_b, fc2_w, fc2_b):
    ei = edge_index.reshape(2, NW, NCH, ECH)
    ews = edge_attr.reshape(NW, NCH, ECH)

    a1, b1, c1 = _run_tc(
        _tc_a_body,
        [jax.ShapeDtypeStruct((N, H1A), jnp.float32),
         jax.ShapeDtypeStruct((N, H1), jnp.float32),
         jax.ShapeDtypeStruct((N, H1), jnp.float32)],
        x, w1a, b1a.reshape(1, H1), w1b, w1c, b1c.reshape(1, H1))

    p1 = _make_sc_edge(H1A, H1A)(a1, ei, ews)

    a2, b2, c2, wdeg = _run_tc(
        _tc_b_body,
        [jax.ShapeDtypeStruct((N, H2A), jnp.float32)]
        + [jax.ShapeDtypeStruct((N, H2), jnp.float32)] * 2
        + [jax.ShapeDtypeStruct((N, 1), jnp.float32)],
        p1, b1, c1, w2a, b2a.reshape(1, H2), w2b, w2c, b2c.reshape(1, H2))

    p2 = _make_sc_edge(H2A, H2)(a2, ei, ews)

    out = _run_tc(
        _tc_c_body,
        jax.ShapeDtypeStruct((N, 10), jnp.float32),
        p2, b2, c2, wdeg, fc1_w, fc1_b.reshape(1, -1), fc2_w, fc2_b.reshape(1, -1))
    return out


# speedup vs baseline: 1.4135x; 1.4135x over previous
import functools

import jax
import jax.numpy as jnp
from jax import lax
from jax.experimental import pallas as pl
from jax.experimental.pallas import tpu as pltpu
from jax.experimental.pallas import tpu_sc as plsc

N = 10000
E = 320000
D = 128
H1 = 32
H1A = 40
H2 = 64
H2A = 72
NPAD = 10240
NW = 32
ECH = 80
NCH = 125
RING = 5
RPS = NPAD // 16


_BCAST_DNUMS = lax.GatherDimensionNumbers(
    offset_dims=(), collapsed_slice_dims=(0,), start_index_map=(0,))


def _elu(v):
    return jnp.where(v > 0, v, jnp.exp(jnp.minimum(v, 0.0)) - 1.0)


def _tc_a_body(x_ref, w1a_ref, b1a_ref, w1b_ref, w1c_ref, b1c_ref,
               a_ref, b_ref, c_ref):
    x = x_ref[...]
    a = jnp.dot(x, w1a_ref[...], preferred_element_type=jnp.float32) + b1a_ref[...]
    a_ref[...] = jnp.concatenate(
        [a, jnp.ones((N, 1), jnp.float32),
         jnp.zeros((N, H1A - H1 - 1), jnp.float32)], axis=1)
    b_ref[...] = jnp.dot(x, w1b_ref[...], preferred_element_type=jnp.float32)
    c_ref[...] = jnp.dot(x, w1c_ref[...], preferred_element_type=jnp.float32) + b1c_ref[...]


def _tc_b_body(p_ref, b1_ref, c1_ref, w2a_ref, b2a_ref, w2b_ref, w2c_ref,
               b2c_ref, a2_ref, b2_ref, c2_ref, wdeg_ref):
    agg = p_ref[0, pl.ds(0, N), pl.ds(0, H1)] + p_ref[1, pl.ds(0, N), pl.ds(0, H1)]
    wdeg = p_ref[0, pl.ds(0, N), pl.ds(H1, 1)] + p_ref[1, pl.ds(0, N), pl.ds(H1, 1)]
    h1 = _elu(agg - b1_ref[...] * wdeg + c1_ref[...])
    a2 = jnp.dot(h1, w2a_ref[...], preferred_element_type=jnp.float32) + b2a_ref[...]
    a2_ref[...] = jnp.concatenate(
        [a2, jnp.zeros((N, H2A - H2), jnp.float32)], axis=1)
    b2_ref[...] = jnp.dot(h1, w2b_ref[...], preferred_element_type=jnp.float32)
    c2_ref[...] = jnp.dot(h1, w2c_ref[...], preferred_element_type=jnp.float32) + b2c_ref[...]
    wdeg_ref[...] = wdeg


def _tc_c_body(p_ref, b2_ref, c2_ref, wdeg_ref, fc1w_ref, fc1b_ref,
               fc2w_ref, fc2b_ref, out_ref):
    agg = (p_ref[0, pl.ds(0, N), pl.ds(0, H2)]
           + p_ref[1, pl.ds(0, N), pl.ds(0, H2)])
    h2 = _elu(agg - b2_ref[...] * wdeg_ref[...] + c2_ref[...])
    h3 = _elu(jnp.dot(h2, fc1w_ref[...], preferred_element_type=jnp.float32) + fc1b_ref[...])
    lg = jnp.dot(h3, fc2w_ref[...], preferred_element_type=jnp.float32) + fc2b_ref[...]
    m = jnp.max(lg, axis=1, keepdims=True)
    out_ref[...] = lg - m - jnp.log(jnp.sum(jnp.exp(lg - m), axis=1, keepdims=True))


def _run_tc(body, out_shapes, *args):
    return pl.pallas_call(body, out_shape=out_shapes)(*args)


@functools.lru_cache(maxsize=None)
def _make_sc_edge(h, zero_offs, scale_offs):
    mesh = plsc.VectorSubcoreMesh(core_axis_name="c", subcore_axis_name="s")

    @functools.partial(
        pl.kernel,
        out_type=jax.ShapeDtypeStruct((2, NPAD, h), jnp.float32),
        mesh=mesh,
        compiler_params=pltpu.CompilerParams(use_tc_tiling_on_sc=False),
        scratch_types=[
            pltpu.VMEM((NCH, ECH), jnp.int32),
            pltpu.VMEM((NCH, ECH), jnp.int32),
            pltpu.VMEM((NCH, ECH), jnp.float32),
            [pltpu.VMEM((ECH, h), jnp.float32)] * RING,
            pltpu.VMEM_SHARED((NPAD, h), jnp.float32),
            [pltpu.SemaphoreType.DMA] * RING,
            [pltpu.SemaphoreType.DMA] * RING,
        ],
    )
    def sc_edge(a_hbm, ei_hbm, ews_hbm, out_hbm,
                src_v, dst_v, ew_v, rows, agg_sh, gsems, ssems):
        c = lax.axis_index("c")
        s = lax.axis_index("s")
        wid = c * 16 + s

        pltpu.sync_copy(ei_hbm.at[0, wid], src_v)
        pltpu.sync_copy(ei_hbm.at[1, wid], dst_v)
        pltpu.sync_copy(ews_hbm.at[wid], ew_v)

        zf = jnp.zeros((16,), jnp.float32)

        def zero_body(r, carry):
            for o in zero_offs:
                rows[0][r, pl.ds(o, 16)] = zf
            return carry

        lax.fori_loop(0, ECH, zero_body, 0)
        for k in range(RPS // ECH):
            pltpu.sync_copy(rows[0], agg_sh.at[pl.ds(s * RPS + k * ECH, ECH)])
        plsc.subcore_barrier()

        def scale(buf, ch):
            def group_body(g, c2):
                vew = ew_v[ch, pl.ds(g * 16, 16)]
                for r16 in range(16):
                    idx = jnp.full((16, 1), r16, jnp.int32)
                    wv = lax.gather(vew, idx, _BCAST_DNUMS, (1,),
                                    mode=lax.GatherScatterMode.PROMISE_IN_BOUNDS)
                    r = g * 16 + r16
                    vals = [buf[r, pl.ds(o, 16)] for o in scale_offs]
                    for o, v in zip(scale_offs, vals):
                        buf[r, pl.ds(o, 16)] = v * wv
                return c2

            lax.fori_loop(0, ECH // 16, group_body, 0)

        pltpu.async_copy(a_hbm.at[src_v.at[0]], rows[0], gsems[0])
        pltpu.async_copy(a_hbm.at[src_v.at[1]], rows[1], gsems[1])

        def ring_body(p, carry):
            for b in range(RING):
                ch = RING * p + b
                bn = (b + 2) % RING

                @pl.when(ch + 2 - RING >= 0)
                def _():
                    pltpu.make_async_copy(
                        rows[bn], agg_sh.at[dst_v.at[ch + 2 - RING]], ssems[bn]
                    ).wait()

                @pl.when(ch + 2 < NCH)
                def _():
                    pltpu.async_copy(
                        a_hbm.at[src_v.at[ch + 2]], rows[bn], gsems[bn])

                pltpu.make_async_copy(
                    a_hbm.at[src_v.at[ch]], rows[b], gsems[b]).wait()
                scale(rows[b], ch)
                pltpu.async_copy(
                    rows[b], agg_sh.at[dst_v.at[ch]], ssems[b], add=True)
            return carry

        lax.fori_loop(0, NCH // RING, ring_body, 0)

        for ch in range(NCH - RING + 2, NCH):
            pltpu.make_async_copy(
                rows[ch % RING], agg_sh.at[dst_v.at[ch]], ssems[ch % RING]
            ).wait()
        plsc.subcore_barrier()

        pltpu.sync_copy(agg_sh.at[pl.ds(s * RPS, RPS)],
                        out_hbm.at[c, pl.ds(s * RPS, RPS)])

    return sc_edge


def kernel(x, edge_index, edge_attr, w1a, b1a, w1b, w1c, b1c,
           w2a, b2a, w2b, w2c, b2c, fc1_w, fc1_b, fc2_w, fc2_b):
    ei = edge_index.reshape(2, NW, NCH, ECH)
    ews = edge_attr.reshape(NW, NCH, ECH)

    a1, b1, c1 = _run_tc(
        _tc_a_body,
        [jax.ShapeDtypeStruct((N, H1A), jnp.float32),
         jax.ShapeDtypeStruct((N, H1), jnp.float32),
         jax.ShapeDtypeStruct((N, H1), jnp.float32)],
        x, w1a, b1a.reshape(1, H1), w1b, w1c, b1c.reshape(1, H1))

    p1 = _make_sc_edge(H1A, (0, 16, 24), (0, 16, 24))(a1, ei, ews)

    a2, b2, c2, wdeg = _run_tc(
        _tc_b_body,
        [jax.ShapeDtypeStruct((N, H2A), jnp.float32)]
        + [jax.ShapeDtypeStruct((N, H2), jnp.float32)] * 2
        + [jax.ShapeDtypeStruct((N, 1), jnp.float32)],
        p1, b1, c1, w2a, b2a.reshape(1, H2), w2b, w2c, b2c.reshape(1, H2))

    p2 = _make_sc_edge(H2A, (0, 16, 32, 48, 56), (0, 16, 32, 48))(a2, ei, ews)

    out = _run_tc(
        _tc_c_body,
        jax.ShapeDtypeStruct((N, 10), jnp.float32),
        p2, b2, c2, wdeg, fc1_w, fc1_b.reshape(1, -1), fc2_w, fc2_b.reshape(1, -1))
    return out
